# baseline (speedup 1.0000x reference)
import jax
import jax.numpy as jnp
from jax import lax
from jax.experimental import pallas as pl
from jax.experimental.pallas import tpu as pltpu
from jax.experimental.pallas import tpu_sc as plsc

N = 10000
E = 320000
H = 128
DEPTH = 3

NC = 2
NS = 16
NW = NC * NS
CHUNK = 64
NCHUNK = 160
EPW = CHUNK * NCHUNK
E_PAD = NW * EPW
N_ACC = 10240
ROWS_PER_SUB = N_ACC // NS
DUMMY = N

_SC_MESH = plsc.VectorSubcoreMesh(core_axis_name="c", subcore_axis_name="s")


NB_S = 3


def _sc_segsum_body(msg_hbm, dst_hbm, zeros_hbm, p_hbm,
                    acc, idx_v, rows_v, lsem, ssem):
    cid = lax.axis_index("c")
    sid = lax.axis_index("s")
    wid = sid * NC + cid

    rbase = sid * ROWS_PER_SUB
    pltpu.sync_copy(zeros_hbm.at[pl.ds(rbase, ROWS_PER_SUB)],
                    acc.at[pl.ds(rbase, ROWS_PER_SUB)])
    plsc.subcore_barrier()

    pltpu.sync_copy(dst_hbm.at[pl.ds(wid * NCHUNK, NCHUNK)], idx_v)

    def _load(j, b):
        base = wid * EPW + j * CHUNK
        pltpu.async_copy(msg_hbm.at[pl.ds(base, CHUNK)], rows_v.at[b],
                         lsem.at[b])

    _load(0, 0)
    _load(1, 1)

    @pl.loop(0, NCHUNK // NB_S + 1)
    def _(jo):
        for b in range(NB_S):
            j = jo * NB_S + b

            @pl.when(j < NCHUNK)
            def _():
                pltpu.make_async_copy(msg_hbm.at[pl.ds(0, CHUNK)],
                                      rows_v.at[b], lsem.at[b]).wait()
                pltpu.async_copy(rows_v.at[b], acc.at[idx_v.at[j]],
                                 ssem.at[b], add=True)

            jn = j + 2
            bn = (b + 2) % NB_S

            @pl.when(jn < NCHUNK)
            def _():
                @pl.when(jn >= NB_S)
                def _():
                    pltpu.make_async_copy(rows_v.at[bn],
                                          acc.at[pl.ds(0, CHUNK)],
                                          ssem.at[bn]).wait()
                _load(jn, bn)

    for b in range(NB_S):
        pltpu.make_async_copy(rows_v.at[b], acc.at[pl.ds(0, CHUNK)],
                              ssem.at[b]).wait()

    plsc.subcore_barrier()
    pltpu.sync_copy(acc.at[pl.ds(rbase, ROWS_PER_SUB)],
                    p_hbm.at[cid, pl.ds(rbase, ROWS_PER_SUB)])


_sc_segsum = pl.kernel(
    _sc_segsum_body,
    out_type=jax.ShapeDtypeStruct((NC, N_ACC, H), jnp.float32),
    mesh=_SC_MESH,
    scratch_types=[
        pltpu.VMEM_SHARED((N_ACC, H), jnp.float32),
        pltpu.VMEM((NCHUNK, CHUNK), jnp.int32),
        pltpu.VMEM((NB_S, CHUNK, H), jnp.float32),
        pltpu.SemaphoreType.DMA((NB_S,)),
        pltpu.SemaphoreType.DMA((NB_S,)),
    ],
)


NB_M = 10


def _sc_mix_body(msg_hbm, namsg_hbm, src_hbm, rev_hbm, d_hbm,
                 sidx_v, ridx_v, dbuf, g1sem, g2sem, wbsem):
    cid = lax.axis_index("c")
    sid = lax.axis_index("s")
    wid = sid * NC + cid

    pltpu.sync_copy(src_hbm.at[pl.ds(wid * NCHUNK, NCHUNK)], sidx_v)
    pltpu.sync_copy(rev_hbm.at[pl.ds(wid * NCHUNK, NCHUNK)], ridx_v)

    def _g1(j, b):
        pltpu.async_copy(msg_hbm.at[ridx_v.at[j]], dbuf.at[b], g1sem.at[b])

    def _g2(j, b):
        pltpu.make_async_copy(msg_hbm.at[pl.ds(0, CHUNK)], dbuf.at[b],
                              g1sem.at[b]).wait()
        pltpu.async_copy(namsg_hbm.at[sidx_v.at[j]], dbuf.at[b], g2sem.at[b],
                         add=True)

    def _wb(j, b):
        pltpu.make_async_copy(namsg_hbm.at[pl.ds(0, CHUNK)], dbuf.at[b],
                              g2sem.at[b]).wait()
        base = wid * EPW + j * CHUNK
        pltpu.async_copy(dbuf.at[b], d_hbm.at[pl.ds(base, CHUNK)],
                         wbsem.at[b])

    @pl.loop(0, (NCHUNK + 8) // NB_M + 1)
    def _(vo):
        for b0 in range(NB_M):
            v = vo * NB_M + b0

            @pl.when(v < NCHUNK)
            def _():
                @pl.when(v >= NB_M)
                def _():
                    pltpu.make_async_copy(dbuf.at[b0],
                                          d_hbm.at[pl.ds(0, CHUNK)],
                                          wbsem.at[b0]).wait()
                _g1(v, b0)

            vg2 = v - 4
            bg2 = (b0 + NB_M - 4) % NB_M

            @pl.when(jnp.logical_and(vg2 >= 0, vg2 < NCHUNK))
            def _():
                _g2(vg2, bg2)

            vwb = v - 8
            bwb = (b0 + NB_M - 8) % NB_M

            @pl.when(jnp.logical_and(vwb >= 0, vwb < NCHUNK))
            def _():
                _wb(vwb, bwb)

    for b in range(NB_M):
        pltpu.make_async_copy(dbuf.at[b], d_hbm.at[pl.ds(0, CHUNK)],
                              wbsem.at[b]).wait()


_sc_mix = pl.kernel(
    _sc_mix_body,
    out_type=jax.ShapeDtypeStruct((E_PAD, H), jnp.float32),
    mesh=_SC_MESH,
    scratch_types=[
        pltpu.VMEM((NCHUNK, CHUNK), jnp.int32),
        pltpu.VMEM((NCHUNK, CHUNK), jnp.int32),
        pltpu.VMEM((NB_M, CHUNK, H), jnp.float32),
        pltpu.SemaphoreType.DMA((NB_M,)),
        pltpu.SemaphoreType.DMA((NB_M,)),
        pltpu.SemaphoreType.DMA((NB_M,)),
    ],
)


BE = 4000
BEP = 4096


def _tc_in_msg_body(fb_ref, wi_ref, msg_ref):
    x = jnp.dot(fb_ref[...], wi_ref[...], preferred_element_type=jnp.float32)
    msg_ref[...] = jnp.maximum(x, 0.0)


def _tc_in_inp_body(fb_ref, wi_ref, inp_ref):
    x = jnp.dot(fb_ref[...], wi_ref[...], preferred_element_type=jnp.float32)
    inp_ref[...] = x.astype(jnp.bfloat16)


def _tc_in_msg(f_bonds, W_i):
    return pl.pallas_call(
        _tc_in_msg_body,
        grid=(E // BE,),
        in_specs=[
            pl.BlockSpec((BE, H), lambda i: (i, 0)),
            pl.BlockSpec((H, H), lambda i: (0, 0)),
        ],
        out_specs=pl.BlockSpec((BE, H), lambda i: (i, 0)),
        out_shape=jax.ShapeDtypeStruct((E_PAD, H), jnp.float32),
    )(f_bonds, W_i)


def _tc_in_inp(f_bonds, W_i):
    return pl.pallas_call(
        _tc_in_inp_body,
        grid=(E // BE,),
        in_specs=[
            pl.BlockSpec((BE, H), lambda i: (i, 0)),
            pl.BlockSpec((H, H), lambda i: (0, 0)),
        ],
        out_specs=pl.BlockSpec((BE, H), lambda i: (i, 0)),
        out_shape=jax.ShapeDtypeStruct((E_PAD, H), jnp.bfloat16),
    )(f_bonds, W_i)


def _tc_negcombine_body(p_ref, na_ref):
    na_ref[...] = -(p_ref[0] + p_ref[1])


def _tc_negcombine(p):
    return pl.pallas_call(
        _tc_negcombine_body,
        grid=(5,),
        in_specs=[pl.BlockSpec((NC, N // 5, H), lambda i: (0, i, 0))],
        out_specs=pl.BlockSpec((N // 5, H), lambda i: (i, 0)),
        out_shape=jax.ShapeDtypeStruct((N, H), jnp.float32),
    )(p)


def _tc_update_body(inp_ref, d_ref, wh_ref, msg_ref):
    x = jnp.dot(d_ref[...], wh_ref[...], preferred_element_type=jnp.float32)
    msg_ref[...] = jnp.maximum(inp_ref[...].astype(jnp.float32) - x, 0.0)


def _tc_update(inp, d, W_h):
    return pl.pallas_call(
        _tc_update_body,
        grid=(E_PAD // BEP,),
        in_specs=[
            pl.BlockSpec((BEP, H), lambda i: (i, 0)),
            pl.BlockSpec((BEP, H), lambda i: (i, 0)),
            pl.BlockSpec((H, H), lambda i: (0, 0)),
        ],
        out_specs=pl.BlockSpec((BEP, H), lambda i: (i, 0)),
        out_shape=jax.ShapeDtypeStruct((E_PAD, H), jnp.float32),
    )(inp, d, W_h)


BN = 2000


def _tc_readout_body(p_ref, fa_ref, wo1_ref, wo2_ref, bo_ref, out_ref):
    a = p_ref[0] + p_ref[1]
    x = jnp.dot(fa_ref[...], wo1_ref[...], preferred_element_type=jnp.float32)
    x = x + jnp.dot(a, wo2_ref[...], preferred_element_type=jnp.float32)
    out_ref[...] = jnp.maximum(x + bo_ref[...], 0.0)


def _tc_readout(p, f_atoms, W_o, b_o):
    wo1 = W_o[:H]
    wo2 = W_o[H:]
    bo = b_o.reshape(1, H)
    return pl.pallas_call(
        _tc_readout_body,
        grid=(N // BN,),
        in_specs=[
            pl.BlockSpec((NC, BN, H), lambda i: (0, i, 0)),
            pl.BlockSpec((BN, H), lambda i: (i, 0)),
            pl.BlockSpec((H, H), lambda i: (0, 0)),
            pl.BlockSpec((H, H), lambda i: (0, 0)),
            pl.BlockSpec((1, H), lambda i: (0, 0)),
        ],
        out_specs=pl.BlockSpec((BN, H), lambda i: (i, 0)),
        out_shape=jax.ShapeDtypeStruct((N, H), jnp.float32),
    )(p, f_atoms, wo1, wo2, bo)


def kernel(f_atoms, f_bonds, edge_src, edge_dst, b2revb, W_i, W_h, W_o, b_o):
    pad = E_PAD - E
    iota = jnp.arange(pad, dtype=jnp.int32)
    src2d = jnp.concatenate(
        [edge_src, iota % N]).reshape(E_PAD // CHUNK, CHUNK)
    dst2d = jnp.concatenate(
        [edge_dst, DUMMY + iota % (N_ACC - N)]).reshape(E_PAD // CHUNK, CHUNK)
    rev2d = jnp.concatenate(
        [b2revb, iota]).reshape(E_PAD // CHUNK, CHUNK)
    zeros = jnp.zeros((N_ACC, H), jnp.float32)

    message = _tc_in_msg(f_bonds, W_i)
    inp = _tc_in_inp(f_bonds, W_i)
    for _ in range(DEPTH - 1):
        p = _sc_segsum(message, dst2d, zeros)
        namsg = _tc_negcombine(p)
        d = _sc_mix(message, namsg, src2d, rev2d)
        message = _tc_update(inp, d, W_h)
    p = _sc_segsum(message, dst2d, zeros)
    return _tc_readout(p, f_atoms, W_o, b_o)


# speedup vs baseline: 1.0772x; 1.0772x over previous
import jax
import jax.numpy as jnp
from jax import lax
from jax.experimental import pallas as pl
from jax.experimental.pallas import tpu as pltpu
from jax.experimental.pallas import tpu_sc as plsc

N = 10000
E = 320000
H = 128
DEPTH = 3

NC = 2
NS = 16
NW = NC * NS
CHUNK = 64
NCHUNK = 160
EPW = CHUNK * NCHUNK
E_PAD = NW * EPW
N_ACC = 10240
ROWS_PER_SUB = N_ACC // NS
DUMMY = N

_SC_MESH = plsc.VectorSubcoreMesh(core_axis_name="c", subcore_axis_name="s")


NB_S = 3


def _sc_segsum_body(msg_hbm, dst_hbm, zeros_hbm, p_hbm,
                    acc, idx_v, rows_v, lsem, ssem):
    cid = lax.axis_index("c")
    sid = lax.axis_index("s")
    wid = sid * NC + cid

    rbase = sid * ROWS_PER_SUB
    pltpu.sync_copy(zeros_hbm.at[pl.ds(rbase, ROWS_PER_SUB)],
                    acc.at[pl.ds(rbase, ROWS_PER_SUB)])
    plsc.subcore_barrier()

    pltpu.sync_copy(dst_hbm.at[pl.ds(wid * NCHUNK, NCHUNK)], idx_v)

    def _load(j, b):
        base = wid * EPW + j * CHUNK
        pltpu.async_copy(msg_hbm.at[pl.ds(base, CHUNK)], rows_v.at[b],
                         lsem.at[b])

    _load(0, 0)
    _load(1, 1)

    @pl.loop(0, NCHUNK // NB_S + 1)
    def _(jo):
        for b in range(NB_S):
            j = jo * NB_S + b

            @pl.when(j < NCHUNK)
            def _():
                pltpu.make_async_copy(msg_hbm.at[pl.ds(0, CHUNK)],
                                      rows_v.at[b], lsem.at[b]).wait()
                pltpu.async_copy(rows_v.at[b], acc.at[idx_v.at[j]],
                                 ssem.at[b], add=True)

            jn = j + 2
            bn = (b + 2) % NB_S

            @pl.when(jn < NCHUNK)
            def _():
                @pl.when(jn >= NB_S)
                def _():
                    pltpu.make_async_copy(rows_v.at[bn],
                                          acc.at[pl.ds(0, CHUNK)],
                                          ssem.at[bn]).wait()
                _load(jn, bn)

    for b in range(NB_S):
        pltpu.make_async_copy(rows_v.at[b], acc.at[pl.ds(0, CHUNK)],
                              ssem.at[b]).wait()

    plsc.subcore_barrier()
    pltpu.sync_copy(acc.at[pl.ds(rbase, ROWS_PER_SUB)],
                    p_hbm.at[cid, pl.ds(rbase, ROWS_PER_SUB)])


_sc_segsum = pl.kernel(
    _sc_segsum_body,
    out_type=jax.ShapeDtypeStruct((NC, N_ACC, H), jnp.float32),
    mesh=_SC_MESH,
    scratch_types=[
        pltpu.VMEM_SHARED((N_ACC, H), jnp.float32),
        pltpu.VMEM((NCHUNK, CHUNK), jnp.int32),
        pltpu.VMEM((NB_S, CHUNK, H), jnp.float32),
        pltpu.SemaphoreType.DMA((NB_S,)),
        pltpu.SemaphoreType.DMA((NB_S,)),
    ],
)


NB_M = 10


def _sc_mix_body(msg_hbm, namsg_hbm, src_hbm, rev_hbm, d_hbm,
                 sidx_v, ridx_v, dbuf, g1sem, g2sem, wbsem):
    cid = lax.axis_index("c")
    sid = lax.axis_index("s")
    wid = sid * NC + cid

    pltpu.sync_copy(src_hbm.at[pl.ds(wid * NCHUNK, NCHUNK)], sidx_v)
    pltpu.sync_copy(rev_hbm.at[pl.ds(wid * NCHUNK, NCHUNK)], ridx_v)

    def _g1(j, b):
        pltpu.async_copy(msg_hbm.at[ridx_v.at[j]], dbuf.at[b], g1sem.at[b])

    def _g2(j, b):
        pltpu.make_async_copy(msg_hbm.at[pl.ds(0, CHUNK)], dbuf.at[b],
                              g1sem.at[b]).wait()
        pltpu.async_copy(namsg_hbm.at[sidx_v.at[j]], dbuf.at[b], g2sem.at[b],
                         add=True)

    def _wb(j, b):
        pltpu.make_async_copy(namsg_hbm.at[pl.ds(0, CHUNK)], dbuf.at[b],
                              g2sem.at[b]).wait()
        base = wid * EPW + j * CHUNK
        pltpu.async_copy(dbuf.at[b], d_hbm.at[pl.ds(base, CHUNK)],
                         wbsem.at[b])

    @pl.loop(0, (NCHUNK + 8) // NB_M + 1)
    def _(vo):
        for b0 in range(NB_M):
            v = vo * NB_M + b0

            @pl.when(v < NCHUNK)
            def _():
                @pl.when(v >= NB_M)
                def _():
                    pltpu.make_async_copy(dbuf.at[b0],
                                          d_hbm.at[pl.ds(0, CHUNK)],
                                          wbsem.at[b0]).wait()
                _g1(v, b0)

            vg2 = v - 4
            bg2 = (b0 + NB_M - 4) % NB_M

            @pl.when(jnp.logical_and(vg2 >= 0, vg2 < NCHUNK))
            def _():
                _g2(vg2, bg2)

            vwb = v - 8
            bwb = (b0 + NB_M - 8) % NB_M

            @pl.when(jnp.logical_and(vwb >= 0, vwb < NCHUNK))
            def _():
                _wb(vwb, bwb)

    for b in range(NB_M):
        pltpu.make_async_copy(dbuf.at[b], d_hbm.at[pl.ds(0, CHUNK)],
                              wbsem.at[b]).wait()


_sc_mix = pl.kernel(
    _sc_mix_body,
    out_type=jax.ShapeDtypeStruct((E_PAD, H), jnp.float32),
    mesh=_SC_MESH,
    scratch_types=[
        pltpu.VMEM((NCHUNK, CHUNK), jnp.int32),
        pltpu.VMEM((NCHUNK, CHUNK), jnp.int32),
        pltpu.VMEM((NB_M, CHUNK, H), jnp.float32),
        pltpu.SemaphoreType.DMA((NB_M,)),
        pltpu.SemaphoreType.DMA((NB_M,)),
        pltpu.SemaphoreType.DMA((NB_M,)),
    ],
)


BE = 8000
BEP = 8192


def _tc_in_body(fb_ref, wi_ref, inp_ref, msg_ref):
    x = jnp.dot(fb_ref[...], wi_ref[...], preferred_element_type=jnp.float32)
    inp_ref[...] = x.astype(jnp.bfloat16)
    msg_ref[...] = jnp.maximum(x, 0.0)


def _tc_in(f_bonds, W_i):
    return pl.pallas_call(
        _tc_in_body,
        grid=(E // BE,),
        in_specs=[
            pl.BlockSpec((BE, H), lambda i: (i, 0)),
            pl.BlockSpec((H, H), lambda i: (0, 0)),
        ],
        out_specs=[
            pl.BlockSpec((BE, H), lambda i: (i, 0)),
            pl.BlockSpec((BE, H), lambda i: (i, 0)),
        ],
        out_shape=[
            jax.ShapeDtypeStruct((E_PAD, H), jnp.bfloat16),
            jax.ShapeDtypeStruct((E_PAD, H), jnp.float32),
        ],
    )(f_bonds, W_i)


def _tc_negcombine_body(p_ref, na_ref):
    na_ref[...] = -(p_ref[0] + p_ref[1])


def _tc_negcombine(p):
    return pl.pallas_call(
        _tc_negcombine_body,
        grid=(5,),
        in_specs=[pl.BlockSpec((NC, N // 5, H), lambda i: (0, i, 0))],
        out_specs=pl.BlockSpec((N // 5, H), lambda i: (i, 0)),
        out_shape=jax.ShapeDtypeStruct((N, H), jnp.float32),
    )(p)


def _tc_update_body(inp_ref, d_ref, wh_ref, msg_ref):
    x = jnp.dot(d_ref[...], wh_ref[...], preferred_element_type=jnp.float32)
    msg_ref[...] = jnp.maximum(inp_ref[...].astype(jnp.float32) - x, 0.0)


def _tc_update(inp, d, W_h):
    return pl.pallas_call(
        _tc_update_body,
        grid=(E_PAD // BEP,),
        in_specs=[
            pl.BlockSpec((BEP, H), lambda i: (i, 0)),
            pl.BlockSpec((BEP, H), lambda i: (i, 0)),
            pl.BlockSpec((H, H), lambda i: (0, 0)),
        ],
        out_specs=pl.BlockSpec((BEP, H), lambda i: (i, 0)),
        out_shape=jax.ShapeDtypeStruct((E_PAD, H), jnp.float32),
    )(inp, d, W_h)


BN = 2000


def _tc_readout_body(p_ref, fa_ref, wo1_ref, wo2_ref, bo_ref, out_ref):
    a = p_ref[0] + p_ref[1]
    x = jnp.dot(fa_ref[...], wo1_ref[...], preferred_element_type=jnp.float32)
    x = x + jnp.dot(a, wo2_ref[...], preferred_element_type=jnp.float32)
    out_ref[...] = jnp.maximum(x + bo_ref[...], 0.0)


def _tc_readout(p, f_atoms, W_o, b_o):
    wo1 = W_o[:H]
    wo2 = W_o[H:]
    bo = b_o.reshape(1, H)
    return pl.pallas_call(
        _tc_readout_body,
        grid=(N // BN,),
        in_specs=[
            pl.BlockSpec((NC, BN, H), lambda i: (0, i, 0)),
            pl.BlockSpec((BN, H), lambda i: (i, 0)),
            pl.BlockSpec((H, H), lambda i: (0, 0)),
            pl.BlockSpec((H, H), lambda i: (0, 0)),
            pl.BlockSpec((1, H), lambda i: (0, 0)),
        ],
        out_specs=pl.BlockSpec((BN, H), lambda i: (i, 0)),
        out_shape=jax.ShapeDtypeStruct((N, H), jnp.float32),
    )(p, f_atoms, wo1, wo2, bo)


def kernel(f_atoms, f_bonds, edge_src, edge_dst, b2revb, W_i, W_h, W_o, b_o):
    pad = E_PAD - E
    iota = jnp.arange(pad, dtype=jnp.int32)
    src2d = jnp.concatenate(
        [edge_src, iota % N]).reshape(E_PAD // CHUNK, CHUNK)
    dst2d = jnp.concatenate(
        [edge_dst, DUMMY + iota % (N_ACC - N)]).reshape(E_PAD // CHUNK, CHUNK)
    rev2d = jnp.concatenate(
        [b2revb, iota]).reshape(E_PAD // CHUNK, CHUNK)
    zeros = jnp.zeros((N_ACC, H), jnp.float32)

    inp, message = _tc_in(f_bonds, W_i)
    for _ in range(DEPTH - 1):
        p = _sc_segsum(message, dst2d, zeros)
        namsg = _tc_negcombine(p)
        d = _sc_mix(message, namsg, src2d, rev2d)
        message = _tc_update(inp, d, W_h)
    p = _sc_segsum(message, dst2d, zeros)
    return _tc_readout(p, f_atoms, W_o, b_o)


# speedup vs baseline: 1.0774x; 1.0002x over previous
import jax
import jax.numpy as jnp
from jax import lax
from jax.experimental import pallas as pl
from jax.experimental.pallas import tpu as pltpu
from jax.experimental.pallas import tpu_sc as plsc

N = 10000
E = 320000
H = 128
DEPTH = 3

NC = 2
NS = 16
NW = NC * NS
CHUNK = 64
NCHUNK = 160
EPW = CHUNK * NCHUNK
E_PAD = NW * EPW
N_ACC = 10240
ROWS_PER_SUB = N_ACC // NS
DUMMY = N

_SC_MESH = plsc.VectorSubcoreMesh(core_axis_name="c", subcore_axis_name="s")


NB_S = 3


def _sc_segsum_body(msg_hbm, dst_hbm, zeros_hbm, p_hbm,
                    acc, idx_v, rows_v, lsem, ssem):
    cid = lax.axis_index("c")
    sid = lax.axis_index("s")
    wid = sid * NC + cid

    rbase = sid * ROWS_PER_SUB
    pltpu.sync_copy(zeros_hbm.at[pl.ds(rbase, ROWS_PER_SUB)],
                    acc.at[pl.ds(rbase, ROWS_PER_SUB)])
    plsc.subcore_barrier()

    pltpu.sync_copy(dst_hbm.at[pl.ds(wid * NCHUNK, NCHUNK)], idx_v)

    def _load(j, b):
        base = wid * EPW + j * CHUNK
        pltpu.async_copy(msg_hbm.at[pl.ds(base, CHUNK)], rows_v.at[b],
                         lsem.at[b])

    _load(0, 0)
    _load(1, 1)

    @pl.loop(0, NCHUNK // NB_S + 1)
    def _(jo):
        for b in range(NB_S):
            j = jo * NB_S + b

            @pl.when(j < NCHUNK)
            def _():
                pltpu.make_async_copy(msg_hbm.at[pl.ds(0, CHUNK)],
                                      rows_v.at[b], lsem.at[b]).wait()
                pltpu.async_copy(rows_v.at[b], acc.at[idx_v.at[j]],
                                 ssem.at[b], add=True)

            jn = j + 2
            bn = (b + 2) % NB_S

            @pl.when(jn < NCHUNK)
            def _():
                @pl.when(jn >= NB_S)
                def _():
                    pltpu.make_async_copy(rows_v.at[bn],
                                          acc.at[pl.ds(0, CHUNK)],
                                          ssem.at[bn]).wait()
                _load(jn, bn)

    for b in range(NB_S):
        pltpu.make_async_copy(rows_v.at[b], acc.at[pl.ds(0, CHUNK)],
                              ssem.at[b]).wait()

    plsc.subcore_barrier()
    pltpu.sync_copy(acc.at[pl.ds(rbase, ROWS_PER_SUB)],
                    p_hbm.at[cid, pl.ds(rbase, ROWS_PER_SUB)])


_sc_segsum = pl.kernel(
    _sc_segsum_body,
    out_type=jax.ShapeDtypeStruct((NC, N_ACC, H), jnp.float32),
    mesh=_SC_MESH,
    scratch_types=[
        pltpu.VMEM_SHARED((N_ACC, H), jnp.float32),
        pltpu.VMEM((NCHUNK, CHUNK), jnp.int32),
        pltpu.VMEM((NB_S, CHUNK, H), jnp.float32),
        pltpu.SemaphoreType.DMA((NB_S,)),
        pltpu.SemaphoreType.DMA((NB_S,)),
    ],
)


NB_M = 10


def _sc_mix_body(msg_hbm, namsg_hbm, src_hbm, rev_hbm, d_hbm,
                 sidx_v, ridx_v, dbuf, g1sem, g2sem, wbsem):
    cid = lax.axis_index("c")
    sid = lax.axis_index("s")
    wid = sid * NC + cid

    pltpu.sync_copy(src_hbm.at[pl.ds(wid * NCHUNK, NCHUNK)], sidx_v)
    pltpu.sync_copy(rev_hbm.at[pl.ds(wid * NCHUNK, NCHUNK)], ridx_v)

    def _g1(j, b):
        pltpu.async_copy(msg_hbm.at[ridx_v.at[j]], dbuf.at[b], g1sem.at[b])

    def _g2(j, b):
        pltpu.make_async_copy(msg_hbm.at[pl.ds(0, CHUNK)], dbuf.at[b],
                              g1sem.at[b]).wait()
        pltpu.async_copy(namsg_hbm.at[sidx_v.at[j]], dbuf.at[b], g2sem.at[b],
                         add=True)

    def _wb(j, b):
        pltpu.make_async_copy(namsg_hbm.at[pl.ds(0, CHUNK)], dbuf.at[b],
                              g2sem.at[b]).wait()
        base = wid * EPW + j * CHUNK
        pltpu.async_copy(dbuf.at[b], d_hbm.at[pl.ds(base, CHUNK)],
                         wbsem.at[b])

    @pl.loop(0, (NCHUNK + 9) // NB_M + 1)
    def _(vo):
        for b0 in range(NB_M):
            v = vo * NB_M + b0

            @pl.when(v < NCHUNK)
            def _():
                @pl.when(v >= NB_M)
                def _():
                    pltpu.make_async_copy(dbuf.at[b0],
                                          d_hbm.at[pl.ds(0, CHUNK)],
                                          wbsem.at[b0]).wait()
                _g1(v, b0)

            vg2 = v - 5
            bg2 = (b0 + NB_M - 5) % NB_M

            @pl.when(jnp.logical_and(vg2 >= 0, vg2 < NCHUNK))
            def _():
                _g2(vg2, bg2)

            vwb = v - 9
            bwb = (b0 + NB_M - 9) % NB_M

            @pl.when(jnp.logical_and(vwb >= 0, vwb < NCHUNK))
            def _():
                _wb(vwb, bwb)

    for b in range(NB_M):
        pltpu.make_async_copy(dbuf.at[b], d_hbm.at[pl.ds(0, CHUNK)],
                              wbsem.at[b]).wait()


_sc_mix = pl.kernel(
    _sc_mix_body,
    out_type=jax.ShapeDtypeStruct((E_PAD, H), jnp.float32),
    mesh=_SC_MESH,
    scratch_types=[
        pltpu.VMEM((NCHUNK, CHUNK), jnp.int32),
        pltpu.VMEM((NCHUNK, CHUNK), jnp.int32),
        pltpu.VMEM((NB_M, CHUNK, H), jnp.float32),
        pltpu.SemaphoreType.DMA((NB_M,)),
        pltpu.SemaphoreType.DMA((NB_M,)),
        pltpu.SemaphoreType.DMA((NB_M,)),
    ],
)


BE = 8000
BEP = 8192


def _tc_in_body(fb_ref, wi_ref, inp_ref, msg_ref):
    x = jnp.dot(fb_ref[...], wi_ref[...], preferred_element_type=jnp.float32)
    inp_ref[...] = x.astype(jnp.bfloat16)
    msg_ref[...] = jnp.maximum(x, 0.0)


def _tc_in(f_bonds, W_i):
    return pl.pallas_call(
        _tc_in_body,
        grid=(E // BE,),
        in_specs=[
            pl.BlockSpec((BE, H), lambda i: (i, 0)),
            pl.BlockSpec((H, H), lambda i: (0, 0)),
        ],
        out_specs=[
            pl.BlockSpec((BE, H), lambda i: (i, 0)),
            pl.BlockSpec((BE, H), lambda i: (i, 0)),
        ],
        out_shape=[
            jax.ShapeDtypeStruct((E_PAD, H), jnp.bfloat16),
            jax.ShapeDtypeStruct((E_PAD, H), jnp.float32),
        ],
    )(f_bonds, W_i)


def _tc_negcombine_body(p_ref, na_ref):
    na_ref[...] = -(p_ref[0] + p_ref[1])


def _tc_negcombine(p):
    return pl.pallas_call(
        _tc_negcombine_body,
        grid=(5,),
        in_specs=[pl.BlockSpec((NC, N // 5, H), lambda i: (0, i, 0))],
        out_specs=pl.BlockSpec((N // 5, H), lambda i: (i, 0)),
        out_shape=jax.ShapeDtypeStruct((N, H), jnp.float32),
    )(p)


def _tc_update_body(inp_ref, d_ref, wh_ref, msg_ref):
    x = jnp.dot(d_ref[...], wh_ref[...], preferred_element_type=jnp.float32)
    msg_ref[...] = jnp.maximum(inp_ref[...].astype(jnp.float32) - x, 0.0)


def _tc_update(inp, d, W_h):
    return pl.pallas_call(
        _tc_update_body,
        grid=(E_PAD // BEP,),
        in_specs=[
            pl.BlockSpec((BEP, H), lambda i: (i, 0)),
            pl.BlockSpec((BEP, H), lambda i: (i, 0)),
            pl.BlockSpec((H, H), lambda i: (0, 0)),
        ],
        out_specs=pl.BlockSpec((BEP, H), lambda i: (i, 0)),
        out_shape=jax.ShapeDtypeStruct((E_PAD, H), jnp.float32),
    )(inp, d, W_h)


BN = 2000


def _tc_readout_body(p_ref, fa_ref, wo1_ref, wo2_ref, bo_ref, out_ref):
    a = p_ref[0] + p_ref[1]
    x = jnp.dot(fa_ref[...], wo1_ref[...], preferred_element_type=jnp.float32)
    x = x + jnp.dot(a, wo2_ref[...], preferred_element_type=jnp.float32)
    out_ref[...] = jnp.maximum(x + bo_ref[...], 0.0)


def _tc_readout(p, f_atoms, W_o, b_o):
    wo1 = W_o[:H]
    wo2 = W_o[H:]
    bo = b_o.reshape(1, H)
    return pl.pallas_call(
        _tc_readout_body,
        grid=(N // BN,),
        in_specs=[
            pl.BlockSpec((NC, BN, H), lambda i: (0, i, 0)),
            pl.BlockSpec((BN, H), lambda i: (i, 0)),
            pl.BlockSpec((H, H), lambda i: (0, 0)),
            pl.BlockSpec((H, H), lambda i: (0, 0)),
            pl.BlockSpec((1, H), lambda i: (0, 0)),
        ],
        out_specs=pl.BlockSpec((BN, H), lambda i: (i, 0)),
        out_shape=jax.ShapeDtypeStruct((N, H), jnp.float32),
    )(p, f_atoms, wo1, wo2, bo)


def kernel(f_atoms, f_bonds, edge_src, edge_dst, b2revb, W_i, W_h, W_o, b_o):
    pad = E_PAD - E
    iota = jnp.arange(pad, dtype=jnp.int32)
    src2d = jnp.concatenate(
        [edge_src, iota % N]).reshape(E_PAD // CHUNK, CHUNK)
    dst2d = jnp.concatenate(
        [edge_dst, DUMMY + iota % (N_ACC - N)]).reshape(E_PAD // CHUNK, CHUNK)
    rev2d = jnp.concatenate(
        [b2revb, iota]).reshape(E_PAD // CHUNK, CHUNK)
    zeros = jnp.zeros((N_ACC, H), jnp.float32)

    inp, message = _tc_in(f_bonds, W_i)
    for _ in range(DEPTH - 1):
        p = _sc_segsum(message, dst2d, zeros)
        namsg = _tc_negcombine(p)
        d = _sc_mix(message, namsg, src2d, rev2d)
        message = _tc_update(inp, d, W_h)
    p = _sc_segsum(message, dst2d, zeros)
    return _tc_readout(p, f_atoms, W_o, b_o)
